# baseline (speedup 1.0000x reference)
import functools

import jax
import jax.numpy as jnp
from jax import lax
from jax.experimental import pallas as pl
from jax.experimental.pallas import tpu as pltpu
from jax.experimental.pallas import tpu_sc as plsc

_MAX_WORDS = 10000
_EMBED = 100
_SEQ = 16
_BATCH = 16384
_OUT = 4

_L = 16
_NC, _NS = 2, 16
_NW = _NC * _NS
_ROWS_W = _BATCH // _NW
_SUB = 128
_NSUB = _ROWS_W // _SUB


def _precompute_body(table_ref, wt_ref, out_ref):
    out_ref[...] = jnp.dot(
        table_ref[...],
        wt_ref[...],
        preferred_element_type=jnp.float32,
        precision=lax.Precision.HIGHEST,
    )


def _precompute(table, wt):
    return pl.pallas_call(
        _precompute_body,
        out_shape=jax.ShapeDtypeStruct((_MAX_WORDS, _SEQ * _L), jnp.float32),
    )(table, wt)


def _sc_body(p_hbm, x_hbm, bias_hbm, out_hbm, xv, idxv, tmp, outv, biasv, sem):
    wid = lax.axis_index("s") * _NC + lax.axis_index("c")
    iota = lax.iota(jnp.int32, _L)
    rowsel = lax.shift_right_logical(iota, 2)
    colsel = lax.bitwise_and(iota, 3)

    pltpu.sync_copy(bias_hbm, biasv)
    bias4 = biasv[...]

    def sub_chunk(c, carry):
        row0 = wid * _ROWS_W + c * _SUB
        pltpu.sync_copy(x_hbm.at[pl.ds(row0, _SUB), :], xv)

        for s in range(_SEQ):
            cols = jnp.full((_L,), s, jnp.int32)
            for v in range(_SUB // _L):
                rows = iota + v * _L
                vals = plsc.load_gather(xv, [rows, cols])
                idxv[s, pl.ds(v * _L, _L)] = vals * _L + s

        copies = []
        for s in range(_SEQ):
            copies.append(pltpu.async_copy(p_hbm.at[idxv.at[s]], tmp.at[s], sem))
        for cp in copies:
            cp.wait()

        def reduce_vreg(v, rcarry):
            acc = bias4
            r = rowsel + v * 4
            for s in range(_SEQ):
                sidx = jnp.full((_L,), s, jnp.int32)
                acc = acc + plsc.load_gather(tmp, [sidx, r, colsel])
            outv[pl.ds(v * _L, _L)] = acc
            return rcarry

        lax.fori_loop(0, _SUB // 4, reduce_vreg, 0)

        pltpu.sync_copy(outv, out_hbm.at[pl.ds(row0 * _OUT, _SUB * _OUT)])
        return carry

    lax.fori_loop(0, _NSUB, sub_chunk, 0)


_sc_call = pl.kernel(
    _sc_body,
    out_type=jax.ShapeDtypeStruct((_BATCH * _OUT,), jnp.float32),
    mesh=plsc.VectorSubcoreMesh(
        core_axis_name="c", subcore_axis_name="s", num_cores=_NC, num_subcores=_NS
    ),
    compiler_params=pltpu.CompilerParams(
        needs_layout_passes=False, use_tc_tiling_on_sc=False
    ),
    scratch_types=[
        pltpu.VMEM((_SUB, _SEQ), jnp.int32),
        pltpu.VMEM((_SEQ, _SUB), jnp.int32),
        pltpu.VMEM((_SEQ, _SUB, _L), jnp.float32),
        pltpu.VMEM((_SUB * _OUT,), jnp.float32),
        pltpu.VMEM((_L,), jnp.float32),
        pltpu.SemaphoreType.DMA,
    ],
)


def kernel(x, table, W, b):
    wt = W.reshape(_OUT, _SEQ, _EMBED).transpose(1, 2, 0)
    wt = jnp.pad(wt, ((0, 0), (0, 0), (0, _L - _OUT)))
    wt2 = wt.transpose(1, 0, 2).reshape(_EMBED, _SEQ * _L)
    p = _precompute(table, wt2).reshape(_SEQ * _MAX_WORDS, _L)
    bias16 = jnp.tile(b, _L // _OUT)
    out_flat = _sc_call(p, x.astype(jnp.int32), bias16)
    return out_flat.reshape(_BATCH, _OUT)


# speedup vs baseline: 1.1373x; 1.1373x over previous
import functools

import jax
import jax.numpy as jnp
from jax import lax
from jax.experimental import pallas as pl
from jax.experimental.pallas import tpu as pltpu
from jax.experimental.pallas import tpu_sc as plsc

_MAX_WORDS = 10000
_EMBED = 100
_SEQ = 16
_BATCH = 16384
_OUT = 4
_ROW = 8

_L = 16
_NC, _NS = 2, 16
_NW = _NC * _NS
_ROWS_W = _BATCH // _NW
_SUB = 128
_NSUB = _ROWS_W // _SUB


def _precompute_body(table_ref, wt_ref, out_ref):
    out_ref[...] = jnp.dot(
        table_ref[...],
        wt_ref[...],
        preferred_element_type=jnp.float32,
        precision=lax.Precision.HIGHEST,
    )


def _precompute(table, wt):
    return pl.pallas_call(
        _precompute_body,
        out_shape=jax.ShapeDtypeStruct((_MAX_WORDS, _SEQ * _ROW), jnp.float32),
    )(table, wt)


def _sc_body(p_hbm, x_hbm, bias_hbm, out_hbm, xv, idxv, tmp, outv, biasv, sem):
    wid = lax.axis_index("s") * _NC + lax.axis_index("c")
    iota = lax.iota(jnp.int32, _L)
    rowsel = lax.shift_right_logical(iota, 2)
    colsel = lax.bitwise_and(iota, 3)

    pltpu.sync_copy(bias_hbm, biasv)
    bias4 = biasv[...]

    def sub_chunk(c, carry):
        row0 = wid * _ROWS_W + c * _SUB
        pltpu.sync_copy(x_hbm.at[pl.ds(row0, _SUB), :], xv)

        for s in range(_SEQ):
            cols = jnp.full((_L,), s, jnp.int32)
            for v in range(_SUB // _L):
                rows = iota + v * _L
                vals = plsc.load_gather(xv, [rows, cols])
                idxv[s, pl.ds(v * _L, _L)] = vals * _L + s

        copies = []
        for s in range(_SEQ):
            copies.append(pltpu.async_copy(p_hbm.at[idxv.at[s]], tmp.at[s], sem))
        for cp in copies:
            cp.wait()

        def reduce_vreg(v, rcarry):
            acc = bias4
            r = rowsel + v * 4
            for s in range(_SEQ):
                sidx = jnp.full((_L,), s, jnp.int32)
                acc = acc + plsc.load_gather(tmp, [sidx, r, colsel])
            outv[pl.ds(v * _L, _L)] = acc
            return rcarry

        lax.fori_loop(0, _SUB // 4, reduce_vreg, 0)

        pltpu.sync_copy(outv, out_hbm.at[pl.ds(row0 * _OUT, _SUB * _OUT)])
        return carry

    lax.fori_loop(0, _NSUB, sub_chunk, 0)


_sc_call = pl.kernel(
    _sc_body,
    out_type=jax.ShapeDtypeStruct((_BATCH * _OUT,), jnp.float32),
    mesh=plsc.VectorSubcoreMesh(
        core_axis_name="c", subcore_axis_name="s", num_cores=_NC, num_subcores=_NS
    ),
    compiler_params=pltpu.CompilerParams(
        needs_layout_passes=False, use_tc_tiling_on_sc=False
    ),
    scratch_types=[
        pltpu.VMEM((_SUB, _SEQ), jnp.int32),
        pltpu.VMEM((_SEQ, _SUB), jnp.int32),
        pltpu.VMEM((_SEQ, _SUB, _ROW), jnp.float32),
        pltpu.VMEM((_SUB * _OUT,), jnp.float32),
        pltpu.VMEM((_L,), jnp.float32),
        pltpu.SemaphoreType.DMA,
    ],
)


def kernel(x, table, W, b):
    wt = W.reshape(_OUT, _SEQ, _EMBED).transpose(1, 2, 0)
    wt = jnp.pad(wt, ((0, 0), (0, 0), (0, _ROW - _OUT)))
    wt2 = wt.transpose(1, 0, 2).reshape(_EMBED, _SEQ * _ROW)
    p = _precompute(table, wt2).reshape(_SEQ * _MAX_WORDS, _ROW)
    bias16 = jnp.tile(b, _L // _OUT)
    out_flat = _sc_call(p, x.astype(jnp.int32), bias16)
    return out_flat.reshape(_BATCH, _OUT)


# speedup vs baseline: 1.4610x; 1.2846x over previous
import functools

import jax
import jax.numpy as jnp
from jax import lax
from jax.experimental import pallas as pl
from jax.experimental.pallas import tpu as pltpu
from jax.experimental.pallas import tpu_sc as plsc

_MAX_WORDS = 10000
_EMBED = 100
_SEQ = 16
_BATCH = 16384
_OUT = 4
_ROW = 8

_L = 16
_NC, _NS = 2, 16
_NW = _NC * _NS
_ROWS_W = _BATCH // _NW
_SUB = 256
_NSUB = _ROWS_W // _SUB


def _precompute_body(table_ref, wt_ref, out_ref):
    out_ref[...] = jnp.dot(
        table_ref[...],
        wt_ref[...],
        preferred_element_type=jnp.float32,
        precision=lax.Precision.HIGHEST,
    )


def _precompute(table, wt):
    return pl.pallas_call(
        _precompute_body,
        out_shape=jax.ShapeDtypeStruct((_MAX_WORDS, _SEQ * _ROW), jnp.float32),
    )(table, wt)


def _sc_body(
    p_hbm, x_hbm, bias_hbm, out_hbm,
    xvs, idxvs, tmps, outv, biasv, sems,
):
    wid = lax.axis_index("s") * _NC + lax.axis_index("c")
    iota = lax.iota(jnp.int32, _L)
    rowsel = lax.shift_right_logical(iota, 2)
    colsel = lax.bitwise_and(iota, 3)

    pltpu.sync_copy(bias_hbm, biasv)
    bias4 = biasv[...]

    def load_prep_fire(c, buf):
        xv, idxv, tmp, sem = xvs[buf], idxvs[buf], tmps[buf], sems[buf]
        row0 = wid * _ROWS_W + c * _SUB
        pltpu.sync_copy(x_hbm.at[:, pl.ds(row0, _SUB)], xv)
        for s in range(_SEQ):
            for v in range(_SUB // _L):
                vals = xv[s, pl.ds(v * _L, _L)]
                idxv[s, pl.ds(v * _L, _L)] = vals * _L + s
        return [
            pltpu.async_copy(p_hbm.at[idxv.at[s]], tmp.at[s], sems[buf])
            for s in range(_SEQ)
        ]

    def reduce_store(c, buf):
        tmp = tmps[buf]

        def reduce_vreg(v, rcarry):
            acc = bias4
            r = rowsel + v * 4
            for s in range(_SEQ):
                sidx = jnp.full((_L,), s, jnp.int32)
                acc = acc + plsc.load_gather(tmp, [sidx, r, colsel])
            outv[pl.ds(v * _L, _L)] = acc
            return rcarry

        lax.fori_loop(0, _SUB // 4, reduce_vreg, 0)
        row0 = wid * _ROWS_W + c * _SUB
        pltpu.sync_copy(outv, out_hbm.at[pl.ds(row0 * _OUT, _SUB * _OUT)])

    inflight = {0: load_prep_fire(0, 0), 1: load_prep_fire(1, 1)}
    for c in range(_NSUB):
        for cp in inflight.pop(c):
            cp.wait()
        reduce_store(c, c % 2)
        if c + 2 < _NSUB:
            inflight[c + 2] = load_prep_fire(c + 2, c % 2)


_sc_call = pl.kernel(
    _sc_body,
    out_type=jax.ShapeDtypeStruct((_BATCH * _OUT,), jnp.float32),
    mesh=plsc.VectorSubcoreMesh(
        core_axis_name="c", subcore_axis_name="s", num_cores=_NC, num_subcores=_NS
    ),
    compiler_params=pltpu.CompilerParams(
        needs_layout_passes=False, use_tc_tiling_on_sc=False
    ),
    scratch_types=[
        [pltpu.VMEM((_SEQ, _SUB), jnp.int32)] * 2,
        [pltpu.VMEM((_SEQ, _SUB), jnp.int32)] * 2,
        [pltpu.VMEM((_SEQ, _SUB, _ROW), jnp.float32)] * 2,
        pltpu.VMEM((_SUB * _OUT,), jnp.float32),
        pltpu.VMEM((_L,), jnp.float32),
        [pltpu.SemaphoreType.DMA] * 2,
    ],
)


def kernel(x, table, W, b):
    wt = W.reshape(_OUT, _SEQ, _EMBED).transpose(1, 2, 0)
    wt = jnp.pad(wt, ((0, 0), (0, 0), (0, _ROW - _OUT)))
    wt2 = wt.transpose(1, 0, 2).reshape(_EMBED, _SEQ * _ROW)
    p = _precompute(table, wt2).reshape(_SEQ * _MAX_WORDS, _ROW)
    bias16 = jnp.tile(b, _L // _OUT)
    out_flat = _sc_call(p, x.astype(jnp.int32).T, bias16)
    return out_flat.reshape(_BATCH, _OUT)


# speedup vs baseline: 1.4973x; 1.0249x over previous
import functools

import jax
import jax.numpy as jnp
from jax import lax
from jax.experimental import pallas as pl
from jax.experimental.pallas import tpu as pltpu
from jax.experimental.pallas import tpu_sc as plsc

_MAX_WORDS = 10000
_EMBED = 100
_SEQ = 16
_BATCH = 16384
_OUT = 4
_ROW = 8

_L = 16
_NC, _NS = 2, 16
_NW = _NC * _NS
_ROWS_W = _BATCH // _NW
_SUB = 128
_NSUB = _ROWS_W // _SUB


def _precompute_body(table_ref, wt_ref, out_ref):
    out_ref[...] = jnp.dot(
        table_ref[...],
        wt_ref[...],
        preferred_element_type=jnp.float32,
        precision=lax.Precision.HIGHEST,
    )


def _precompute(table, wt):
    return pl.pallas_call(
        _precompute_body,
        out_shape=jax.ShapeDtypeStruct((_MAX_WORDS, _SEQ * _ROW), jnp.float32),
    )(table, wt)


def _sc_body(
    p_hbm, x_hbm, bias_hbm, out_hbm,
    xvs, idxvs, accs, outv, biasv, sems,
):
    wid = lax.axis_index("s") * _NC + lax.axis_index("c")
    iota = lax.iota(jnp.int32, _L)
    rowsel = lax.shift_right_logical(iota, 2)
    colsel = lax.bitwise_and(iota, 3)

    pltpu.sync_copy(bias_hbm, biasv)
    bias4 = biasv[...]

    zero = jnp.zeros((_L,), jnp.float32)

    def load_prep_fire(c, buf):
        xv, idxv, acc = xvs[buf], idxvs[buf], accs[buf]
        row0 = wid * _ROWS_W + c * _SUB

        def zero_vreg(v, zcarry):
            plsc.store_scatter(acc, [rowsel + v * 4, colsel], zero)
            return zcarry

        lax.fori_loop(0, _SUB // 4, zero_vreg, 0)
        pltpu.sync_copy(x_hbm.at[:, pl.ds(row0, _SUB)], xv)
        for s in range(_SEQ):
            for v in range(_SUB // _L):
                vals = xv[s, pl.ds(v * _L, _L)]
                idxv[s, pl.ds(v * _L, _L)] = vals * _L + s
        return [
            pltpu.async_copy(p_hbm.at[idxv.at[s]], acc, sems[buf], add=True)
            for s in range(_SEQ)
        ]

    def pack_store(c, buf):
        acc = accs[buf]

        def pack_vreg(v, rcarry):
            got = plsc.load_gather(acc, [rowsel + v * 4, colsel])
            outv[pl.ds(v * _L, _L)] = got + bias4
            return rcarry

        lax.fori_loop(0, _SUB // 4, pack_vreg, 0)
        row0 = wid * _ROWS_W + c * _SUB
        pltpu.sync_copy(outv, out_hbm.at[pl.ds(row0 * _OUT, _SUB * _OUT)])

    inflight = {0: load_prep_fire(0, 0), 1: load_prep_fire(1, 1)}
    for c in range(_NSUB):
        for cp in inflight.pop(c):
            cp.wait()
        pack_store(c, c % 2)
        if c + 2 < _NSUB:
            inflight[c + 2] = load_prep_fire(c + 2, c % 2)


_sc_call = pl.kernel(
    _sc_body,
    out_type=jax.ShapeDtypeStruct((_BATCH * _OUT,), jnp.float32),
    mesh=plsc.VectorSubcoreMesh(
        core_axis_name="c", subcore_axis_name="s", num_cores=_NC, num_subcores=_NS
    ),
    compiler_params=pltpu.CompilerParams(
        needs_layout_passes=False, use_tc_tiling_on_sc=False
    ),
    scratch_types=[
        [pltpu.VMEM((_SEQ, _SUB), jnp.int32)] * 2,
        [pltpu.VMEM((_SEQ, _SUB), jnp.int32)] * 2,
        [pltpu.VMEM((_SUB, _ROW), jnp.float32)] * 2,
        pltpu.VMEM((_SUB * _OUT,), jnp.float32),
        pltpu.VMEM((_L,), jnp.float32),
        [pltpu.SemaphoreType.DMA] * 2,
    ],
)


def kernel(x, table, W, b):
    wt = W.reshape(_OUT, _SEQ, _EMBED).transpose(1, 2, 0)
    wt = jnp.pad(wt, ((0, 0), (0, 0), (0, _ROW - _OUT)))
    wt2 = wt.transpose(1, 0, 2).reshape(_EMBED, _SEQ * _ROW)
    p = _precompute(table, wt2).reshape(_SEQ * _MAX_WORDS, _ROW)
    bias16 = jnp.tile(b, _L // _OUT)
    out_flat = _sc_call(p, x.astype(jnp.int32).T, bias16)
    return out_flat.reshape(_BATCH, _OUT)


# speedup vs baseline: 1.5083x; 1.0073x over previous
import functools

import jax
import jax.numpy as jnp
from jax import lax
from jax.experimental import pallas as pl
from jax.experimental.pallas import tpu as pltpu
from jax.experimental.pallas import tpu_sc as plsc

_MAX_WORDS = 10000
_EMBED = 100
_SEQ = 16
_BATCH = 16384
_OUT = 4
_ROW = 8

_L = 16
_NC, _NS = 2, 16
_NW = _NC * _NS
_ROWS_W = _BATCH // _NW
_SUB = 128
_NSUB = _ROWS_W // _SUB


def _precompute_body(table_ref, wt_ref, out_ref):
    out_ref[...] = jnp.dot(
        table_ref[...],
        wt_ref[...],
        preferred_element_type=jnp.float32,
        precision=lax.Precision.HIGHEST,
    )


def _precompute(table, wt):
    return pl.pallas_call(
        _precompute_body,
        out_shape=jax.ShapeDtypeStruct((_MAX_WORDS, _SEQ * _ROW), jnp.float32),
    )(table, wt)


def _sc_body(
    p_hbm, x_hbm, bias_hbm, out_hbm,
    xvs, idxvs, accs, outv, biasv, sems,
):
    wid = lax.axis_index("s") * _NC + lax.axis_index("c")
    iota = lax.iota(jnp.int32, _L)
    rowsel = lax.shift_right_logical(iota, 2)
    colsel = lax.bitwise_and(iota, 3)

    pltpu.sync_copy(bias_hbm, biasv)
    bias4 = biasv[...]

    zero = jnp.zeros((_L,), jnp.float32)

    def load_prep_fire(c, buf):
        xv, idxv, acc = xvs[buf], idxvs[buf], accs[buf]
        row0 = wid * _ROWS_W + c * _SUB

        def zero_vreg(v, zcarry):
            plsc.store_scatter(acc, [rowsel + v * 4, colsel], zero)
            return zcarry

        lax.fori_loop(0, _SUB // 4, zero_vreg, 0)
        pltpu.sync_copy(x_hbm.at[:, pl.ds(row0, _SUB)], xv)
        for s in range(_SEQ):
            for v in range(_SUB // _L):
                vals = xv[s, pl.ds(v * _L, _L)]
                idxv[s, pl.ds(v * _L, _L)] = vals * _L + s
        return [
            pltpu.async_copy(p_hbm.at[idxv.at[s]], acc, sems[buf], add=True)
            for s in range(_SEQ)
        ]

    def pack_store(c, buf):
        acc = accs[buf]

        def pack_vreg(v, rcarry):
            got = plsc.load_gather(acc, [rowsel + v * 4, colsel])
            outv[pl.ds(v * _L, _L)] = got + bias4
            return rcarry

        lax.fori_loop(0, _SUB // 4, pack_vreg, 0)
        row0 = wid * _ROWS_W + c * _SUB
        pltpu.sync_copy(outv, out_hbm.at[pl.ds(row0 * _OUT, _SUB * _OUT)])

    inflight = [load_prep_fire(c, c) for c in range(_NSUB)]
    for c in range(_NSUB):
        for cp in inflight[c]:
            cp.wait()
        pack_store(c, c)


_sc_call = pl.kernel(
    _sc_body,
    out_type=jax.ShapeDtypeStruct((_BATCH * _OUT,), jnp.float32),
    mesh=plsc.VectorSubcoreMesh(
        core_axis_name="c", subcore_axis_name="s", num_cores=_NC, num_subcores=_NS
    ),
    compiler_params=pltpu.CompilerParams(
        needs_layout_passes=False, use_tc_tiling_on_sc=False
    ),
    scratch_types=[
        [pltpu.VMEM((_SEQ, _SUB), jnp.int32)] * _NSUB,
        [pltpu.VMEM((_SEQ, _SUB), jnp.int32)] * _NSUB,
        [pltpu.VMEM((_SUB, _ROW), jnp.float32)] * _NSUB,
        pltpu.VMEM((_SUB * _OUT,), jnp.float32),
        pltpu.VMEM((_L,), jnp.float32),
        [pltpu.SemaphoreType.DMA] * _NSUB,
    ],
)


def kernel(x, table, W, b):
    wt = W.reshape(_OUT, _SEQ, _EMBED).transpose(1, 2, 0)
    wt = jnp.pad(wt, ((0, 0), (0, 0), (0, _ROW - _OUT)))
    wt2 = wt.transpose(1, 0, 2).reshape(_EMBED, _SEQ * _ROW)
    p = _precompute(table, wt2).reshape(_SEQ * _MAX_WORDS, _ROW)
    bias16 = jnp.tile(b, _L // _OUT)
    out_flat = _sc_call(p, x.astype(jnp.int32).T, bias16)
    return out_flat.reshape(_BATCH, _OUT)


# speedup vs baseline: 1.5923x; 1.0557x over previous
import functools

import jax
import jax.numpy as jnp
from jax import lax
from jax.experimental import pallas as pl
from jax.experimental.pallas import tpu as pltpu
from jax.experimental.pallas import tpu_sc as plsc

_MAX_WORDS = 10000
_EMBED = 100
_SEQ = 16
_BATCH = 16384
_OUT = 4
_ROW = 8

_L = 16
_NC, _NS = 2, 16
_NW = _NC * _NS
_ROWS_W = _BATCH // _NW
_SUB = 128
_NSUB = _ROWS_W // _SUB


def _precompute_body(table_ref, wt_ref, out_ref):
    out_ref[...] = jnp.dot(
        table_ref[...],
        wt_ref[...],
        preferred_element_type=jnp.float32,
    )


def _precompute(table, wt):
    return pl.pallas_call(
        _precompute_body,
        out_shape=jax.ShapeDtypeStruct((_MAX_WORDS, _SEQ * _ROW), jnp.float32),
    )(table, wt)


def _sc_body(
    p_hbm, x_hbm, bias_hbm, out_hbm,
    xvs, idxvs, accs, outv, biasv, sems,
):
    wid = lax.axis_index("s") * _NC + lax.axis_index("c")
    iota = lax.iota(jnp.int32, _L)
    rowsel = lax.shift_right_logical(iota, 2)
    colsel = lax.bitwise_and(iota, 3)

    pltpu.sync_copy(bias_hbm, biasv)
    bias4 = biasv[...]

    zero = jnp.zeros((_L,), jnp.float32)

    def load_prep_fire(c, buf):
        xv, idxv, acc = xvs[buf], idxvs[buf], accs[buf]
        row0 = wid * _ROWS_W + c * _SUB

        def zero_vreg(v, zcarry):
            plsc.store_scatter(acc, [rowsel + v * 4, colsel], zero)
            return zcarry

        lax.fori_loop(0, _SUB // 4, zero_vreg, 0)
        pltpu.sync_copy(x_hbm.at[:, pl.ds(row0, _SUB)], xv)
        for s in range(_SEQ):
            for v in range(_SUB // _L):
                vals = xv[s, pl.ds(v * _L, _L)]
                idxv[s, pl.ds(v * _L, _L)] = vals * _L + s
        return [
            pltpu.async_copy(p_hbm.at[idxv.at[s]], acc, sems[buf], add=True)
            for s in range(_SEQ)
        ]

    def pack_store(c, buf):
        acc = accs[buf]

        def pack_vreg(v, rcarry):
            got = plsc.load_gather(acc, [rowsel + v * 4, colsel])
            outv[pl.ds(v * _L, _L)] = got + bias4
            return rcarry

        lax.fori_loop(0, _SUB // 4, pack_vreg, 0)
        row0 = wid * _ROWS_W + c * _SUB
        pltpu.sync_copy(outv, out_hbm.at[pl.ds(row0 * _OUT, _SUB * _OUT)])

    inflight = [load_prep_fire(c, c) for c in range(_NSUB)]
    for c in range(_NSUB):
        for cp in inflight[c]:
            cp.wait()
        pack_store(c, c)


_sc_call = pl.kernel(
    _sc_body,
    out_type=jax.ShapeDtypeStruct((_BATCH * _OUT,), jnp.float32),
    mesh=plsc.VectorSubcoreMesh(
        core_axis_name="c", subcore_axis_name="s", num_cores=_NC, num_subcores=_NS
    ),
    compiler_params=pltpu.CompilerParams(
        needs_layout_passes=False, use_tc_tiling_on_sc=False
    ),
    scratch_types=[
        [pltpu.VMEM((_SEQ, _SUB), jnp.int32)] * _NSUB,
        [pltpu.VMEM((_SEQ, _SUB), jnp.int32)] * _NSUB,
        [pltpu.VMEM((_SUB, _ROW), jnp.float32)] * _NSUB,
        pltpu.VMEM((_SUB * _OUT,), jnp.float32),
        pltpu.VMEM((_L,), jnp.float32),
        [pltpu.SemaphoreType.DMA] * _NSUB,
    ],
)


def kernel(x, table, W, b):
    wt = W.reshape(_OUT, _SEQ, _EMBED).transpose(1, 2, 0)
    wt = jnp.pad(wt, ((0, 0), (0, 0), (0, _ROW - _OUT)))
    wt2 = wt.transpose(1, 0, 2).reshape(_EMBED, _SEQ * _ROW)
    p = _precompute(table, wt2).reshape(_SEQ * _MAX_WORDS, _ROW)
    bias16 = jnp.tile(b, _L // _OUT)
    out_flat = _sc_call(p, x.astype(jnp.int32).T, bias16)
    return out_flat.reshape(_BATCH, _OUT)
